# baseline (speedup 1.0000x reference)
import functools

import jax
import jax.numpy as jnp
from jax import lax
from jax.experimental import pallas as pl
from jax.experimental.pallas import tpu as pltpu
from jax.experimental.pallas import tpu_sc as plsc

_LANES = 128
_NW = 32
_VOCAB_DIM = 64


def _convert_table_sc(table_t):
    d, v = table_t.shape
    n_tiles = -(-v // _LANES)
    full = v // _LANES
    rem = v - full * _LANES
    per_w = -(-n_tiles // _NW)
    mesh = plsc.VectorSubcoreMesh(core_axis_name="c", subcore_axis_name="s")

    @functools.partial(
        pl.kernel, mesh=mesh,
        out_type=jax.ShapeDtypeStruct((v * d,), jnp.float32),
        scratch_types=[
            pltpu.VMEM((d, _LANES), jnp.float32),
            pltpu.VMEM((_LANES * d,), jnp.float32),
        ],
        compiler_params=pltpu.CompilerParams(
            use_tc_tiling_on_sc=True, needs_layout_passes=False),
    )
    def conv_kernel(tin, tout, cols, trflat):
        wid = lax.axis_index("s") * 2 + lax.axis_index("c")
        iota16 = lax.iota(jnp.int32, 16)

        def unit(i, carry):
            vt = wid * per_w + i

            @pl.when(vt < n_tiles)
            def _():
                pltpu.sync_copy(tin.at[:, pl.ds(vt * _LANES, _LANES)], cols)

                def tr_body(c0, c2):
                    dj = jnp.bitwise_and(c0 + iota16, d - 1)
                    for lg in range(_LANES // 16):
                        lv = iota16 + 16 * lg
                        vec = plsc.load_gather(cols, [dj, lv])
                        plsc.store_scatter(trflat, [lv * d + dj], vec)
                    return c2
                lax.fori_loop(0, d, tr_body, 0, unroll=4)

                @pl.when(vt < full)
                def _():
                    pltpu.sync_copy(
                        trflat, tout.at[pl.ds(vt * _LANES * d, _LANES * d)])

                @pl.when(vt == full)
                def _():
                    pltpu.sync_copy(trflat.at[pl.ds(0, rem * d)],
                                    tout.at[pl.ds(full * _LANES * d, rem * d)])
            return carry

        lax.fori_loop(0, per_w, unit, 0)

    return conv_kernel(table_t)


def _gather_transpose_sc(ids_lin, table):
    (t_total,) = ids_lin.shape
    _, d = table.shape
    n_units = t_total // _LANES
    units_per_w = n_units // _NW
    mesh = plsc.VectorSubcoreMesh(core_axis_name="c", subcore_axis_name="s")

    @functools.partial(
        pl.kernel,
        mesh=mesh,
        out_type=jax.ShapeDtypeStruct((t_total // (8 * _LANES), d, 8, _LANES),
                                      jnp.float32),
        scratch_types=[
            pltpu.VMEM((units_per_w * _LANES,), jnp.int32),
            pltpu.VMEM((_LANES, d), jnp.float32),
            pltpu.VMEM((_LANES, d), jnp.float32),
            pltpu.VMEM((d, _LANES), jnp.float32),
            pltpu.VMEM((d, _LANES), jnp.float32),
            pltpu.SemaphoreType.DMA,
            pltpu.SemaphoreType.DMA,
            pltpu.SemaphoreType.DMA,
            pltpu.SemaphoreType.DMA,
        ],
        compiler_params=pltpu.CompilerParams(
            use_tc_tiling_on_sc=False,
            needs_layout_passes=False,
        ),
    )
    def gather_kernel(table_hbm, idx_hbm, out_hbm, idx_all,
                      rows_a, rows_b, tr_a, tr_b, sg_a, sg_b, sw_a, sw_b):
        wid = lax.axis_index("s") * 2 + lax.axis_index("c")
        base_u = wid * units_per_w

        pltpu.sync_copy(
            idx_hbm.at[pl.ds(pl.multiple_of(base_u * _LANES, _LANES),
                             units_per_w * _LANES)],
            idx_all)

        def start_gather(i, rows, sem):
            pltpu.async_copy(
                table_hbm.at[idx_all.at[pl.ds(i * _LANES, _LANES)]], rows, sem)

        iota16 = lax.iota(jnp.int32, 16)

        def transpose(rows, tr):
            def tr_body(c0, c2):
                dj = jnp.bitwise_and(c0 + iota16, d - 1)
                for lg in range(_LANES // 16):
                    lv = iota16 + (16 * lg)
                    vec = plsc.load_gather(rows, [lv, dj])
                    plsc.store_scatter(tr, [dj, lv], vec)
                return c2
            lax.fori_loop(0, d, tr_body, 0, unroll=4)

        def out_window(i):
            u = base_u + i
            s = (u // 64) * 8 + u % 8
            bt = (u // 8) % 8
            return out_hbm.at[s, :, bt, :]

        def start_write(i, tr, sem):
            pltpu.async_copy(tr, out_window(i), sem)

        def wait_gather(rows, sem):
            pltpu.make_async_copy(table_hbm.at[idx_all.at[pl.ds(0, _LANES)]],
                                  rows, sem).wait()

        def wait_write(i, tr, sem):
            pltpu.make_async_copy(tr, out_window(i), sem).wait()

        start_gather(0, rows_a, sg_a)

        def step(k, carry):
            i0 = 2 * k
            i1 = 2 * k + 1
            start_gather(i1, rows_b, sg_b)
            wait_gather(rows_a, sg_a)

            @pl.when(k > 0)
            def _():
                wait_write(i0, tr_a, sw_a)
            transpose(rows_a, tr_a)
            start_write(i0, tr_a, sw_a)

            @pl.when(k < (units_per_w // 2 - 1))
            def _():
                start_gather(i0 + 2, rows_a, sg_a)
            wait_gather(rows_b, sg_b)

            @pl.when(k > 0)
            def _():
                wait_write(i1, tr_b, sw_b)
            transpose(rows_b, tr_b)
            start_write(i1, tr_b, sw_b)
            return carry

        lax.fori_loop(0, units_per_w // 2, step, 0)
        wait_write(0, tr_a, sw_a)
        wait_write(0, tr_b, sw_b)

    return gather_kernel(table, ids_lin)


def _rotate_tc(x_lin, a3, block_s):
    n_s, d, two_bt, lanes = x_lin.shape
    bt = two_bt // 2

    def body(x_ref, a_ref, o_ref):
        v = x_ref[...]
        a = a_ref[...]
        asq = a * a
        recip = 1.0 / (1.0 + asq)
        c8 = (1.0 - asq) * recip
        s8 = (2.0 * a) * recip
        shape16 = (block_s, d, two_bt, lanes)
        c16 = jnp.broadcast_to(c8[:, :, :, None, :],
                               (block_s, d, bt, 2, lanes)).reshape(shape16)
        s16 = jnp.broadcast_to(s8[:, :, :, None, :],
                               (block_s, d, bt, 2, lanes)).reshape(shape16)
        i_idx = lax.broadcasted_iota(jnp.int32, shape16, 2)
        even = (i_idx % 2) == 0
        w = jnp.where(even, jnp.roll(v, -1, axis=2), jnp.roll(v, 1, axis=2))
        s_signed = jnp.where(even, -s16, s16)
        o_ref[...] = v * c16 + w * s_signed

    return pl.pallas_call(
        body,
        grid=(n_s // block_s,),
        in_specs=[
            pl.BlockSpec((block_s, d, two_bt, lanes), lambda i: (i, 0, 0, 0)),
            pl.BlockSpec((block_s, d, bt, lanes), lambda i: (i, 0, 0, 0)),
        ],
        out_specs=pl.BlockSpec((block_s, d, two_bt, lanes),
                               lambda i: (i, 0, 0, 0)),
        out_shape=jax.ShapeDtypeStruct((n_s, d, two_bt, lanes), jnp.float32),
        compiler_params=pltpu.CompilerParams(
            dimension_semantics=("arbitrary",),
        ),
    )(x_lin, a3)


def kernel(x, affix_ids, rotation_params):
    b, s, d, _ = x.shape
    nbt = b // _LANES
    x_lin = (x.transpose(1, 2, 3, 0)
              .reshape(s, d, 2, nbt, _LANES)
              .transpose(0, 1, 3, 2, 4)
              .reshape(s, d, 2 * nbt, _LANES))
    ids_lin = (affix_ids.astype(jnp.int32).T
               .reshape(s // 8, 8, b // _LANES, _LANES)
               .transpose(0, 2, 1, 3)
               .reshape(-1))
    table_lin = _convert_table_sc(rotation_params.T).reshape(
        rotation_params.shape)
    a3 = _gather_transpose_sc(ids_lin, table_lin)
    out_lin = _rotate_tc(x_lin, a3, block_s=4)
    out = (out_lin.reshape(s, d, nbt, 2, _LANES)
                  .transpose(2, 4, 0, 1, 3)
                  .reshape(b, s, d, 2))
    return out


# speedup vs baseline: 1.2213x; 1.2213x over previous
import functools

import jax
import jax.numpy as jnp
from jax import lax
from jax.experimental import pallas as pl
from jax.experimental.pallas import tpu as pltpu
from jax.experimental.pallas import tpu_sc as plsc

_LANES = 128
_NW = 32
_VOCAB_DIM = 64


def _gather_transpose_sc(ids_lin, table):
    (t_total,) = ids_lin.shape
    _, d = table.shape
    n_units = t_total // _LANES
    units_per_w = n_units // _NW
    mesh = plsc.VectorSubcoreMesh(core_axis_name="c", subcore_axis_name="s")

    @functools.partial(
        pl.kernel,
        mesh=mesh,
        out_type=jax.ShapeDtypeStruct((t_total // (8 * _LANES), d, 16, _LANES),
                                      jnp.float32),
        scratch_types=[
            pltpu.VMEM((units_per_w * _LANES,), jnp.int32),
            pltpu.VMEM((_LANES, d), jnp.float32),
            pltpu.VMEM((_LANES, d), jnp.float32),
            pltpu.VMEM((d, 2, _LANES), jnp.float32),
            pltpu.VMEM((d, 2, _LANES), jnp.float32),
            pltpu.SemaphoreType.DMA,
            pltpu.SemaphoreType.DMA,
            pltpu.SemaphoreType.DMA,
            pltpu.SemaphoreType.DMA,
        ],
        compiler_params=pltpu.CompilerParams(
            use_tc_tiling_on_sc=False,
            needs_layout_passes=False,
        ),
    )
    def gather_kernel(table_hbm, idx_hbm, out_hbm, idx_all,
                      rows_a, rows_b, tr_a, tr_b, sg_a, sg_b, sw_a, sw_b):
        wid = lax.axis_index("s") * 2 + lax.axis_index("c")
        base_u = wid * units_per_w

        pltpu.sync_copy(
            idx_hbm.at[pl.ds(pl.multiple_of(base_u * _LANES, _LANES),
                             units_per_w * _LANES)],
            idx_all)

        def start_gather(i, rows, sem):
            pltpu.async_copy(
                table_hbm.at[idx_all.at[pl.ds(i * _LANES, _LANES)]], rows, sem)

        iota16 = lax.iota(jnp.int32, 16)
        zero16 = jnp.zeros((16,), jnp.int32)
        one16 = jnp.ones((16,), jnp.int32)

        def transpose(rows, tr):
            def tr_body(c0, c2):
                dj = jnp.bitwise_and(c0 + iota16, d - 1)
                for lg in range(_LANES // 16):
                    lv = iota16 + (16 * lg)
                    vec = plsc.load_gather(rows, [lv, dj])
                    plsc.store_scatter(tr, [dj, zero16, lv], -vec)
                    plsc.store_scatter(tr, [dj, one16, lv], vec)
                return c2
            lax.fori_loop(0, d, tr_body, 0, unroll=4)

        def out_window(i):
            u = base_u + i
            s = (u // 64) * 8 + u % 8
            bt = (u // 8) % 8
            return out_hbm.at[s, :, pl.ds(2 * bt, 2), :]

        def start_write(i, tr, sem):
            pltpu.async_copy(tr, out_window(i), sem)

        def wait_gather(rows, sem):
            pltpu.make_async_copy(table_hbm.at[idx_all.at[pl.ds(0, _LANES)]],
                                  rows, sem).wait()

        def wait_write(i, tr, sem):
            pltpu.make_async_copy(tr, out_window(i), sem).wait()

        start_gather(0, rows_a, sg_a)

        def step(k, carry):
            i0 = 2 * k
            i1 = 2 * k + 1
            start_gather(i1, rows_b, sg_b)
            wait_gather(rows_a, sg_a)

            @pl.when(k > 0)
            def _():
                wait_write(i0, tr_a, sw_a)
            transpose(rows_a, tr_a)
            start_write(i0, tr_a, sw_a)

            @pl.when(k < (units_per_w // 2 - 1))
            def _():
                start_gather(i0 + 2, rows_a, sg_a)
            wait_gather(rows_b, sg_b)

            @pl.when(k > 0)
            def _():
                wait_write(i1, tr_b, sw_b)
            transpose(rows_b, tr_b)
            start_write(i1, tr_b, sw_b)
            return carry

        lax.fori_loop(0, units_per_w // 2, step, 0)
        wait_write(0, tr_a, sw_a)
        wait_write(0, tr_b, sw_b)

    return gather_kernel(table, ids_lin)


def _rotate_tc(x_lin, a16, block_s):
    n_s, d, two_bt, lanes = x_lin.shape
    bt = two_bt // 2

    def body(x_ref, a_ref, o_ref):
        v = x_ref[...]
        asgn = a_ref[...]
        t = 2.0 / (1.0 + asgn * asgn)
        c16 = t - 1.0
        ss16 = asgn * t
        i_idx = lax.broadcasted_iota(jnp.int32, v.shape, 2)
        even = (i_idx % 2) == 0
        w = jnp.where(even, jnp.roll(v, -1, axis=2), jnp.roll(v, 1, axis=2))
        o_ref[...] = v * c16 + w * ss16

    return pl.pallas_call(
        body,
        grid=(n_s // block_s,),
        in_specs=[
            pl.BlockSpec((block_s, d, two_bt, lanes), lambda i: (i, 0, 0, 0)),
            pl.BlockSpec((block_s, d, two_bt, lanes), lambda i: (i, 0, 0, 0)),
        ],
        out_specs=pl.BlockSpec((block_s, d, two_bt, lanes),
                               lambda i: (i, 0, 0, 0)),
        out_shape=jax.ShapeDtypeStruct((n_s, d, two_bt, lanes), jnp.float32),
        compiler_params=pltpu.CompilerParams(
            dimension_semantics=("arbitrary",),
        ),
    )(x_lin, a16)


def kernel(x, affix_ids, rotation_params):
    b, s, d, _ = x.shape
    nbt = b // _LANES
    x_lin = (x.transpose(1, 2, 3, 0)
              .reshape(s, d, 2, nbt, _LANES)
              .transpose(0, 1, 3, 2, 4)
              .reshape(s, d, 2 * nbt, _LANES))
    ids_lin = (affix_ids.astype(jnp.int32).T
               .reshape(s // 8, 8, b // _LANES, _LANES)
               .transpose(0, 2, 1, 3)
               .reshape(-1))
    a3 = _gather_transpose_sc(ids_lin, rotation_params)
    out_lin = _rotate_tc(x_lin, a3, block_s=4)
    out = (out_lin.reshape(s, d, nbt, 2, _LANES)
                  .transpose(2, 4, 0, 1, 3)
                  .reshape(b, s, d, 2))
    return out


# speedup vs baseline: 1.2282x; 1.0056x over previous
import functools

import jax
import jax.numpy as jnp
from jax import lax
from jax.experimental import pallas as pl
from jax.experimental.pallas import tpu as pltpu
from jax.experimental.pallas import tpu_sc as plsc

_LANES = 128
_NW = 32
_VOCAB_DIM = 64


def _gather_transpose_sc(ids_lin, table):
    (t_total,) = ids_lin.shape
    _, d = table.shape
    n_units = t_total // _LANES
    units_per_w = n_units // _NW
    mesh = plsc.VectorSubcoreMesh(core_axis_name="c", subcore_axis_name="s")

    @functools.partial(
        pl.kernel,
        mesh=mesh,
        out_type=jax.ShapeDtypeStruct((t_total // (8 * _LANES), d, 16, _LANES),
                                      jnp.float32),
        scratch_types=[
            pltpu.VMEM((units_per_w * _LANES,), jnp.int32),
            pltpu.VMEM((_LANES, d), jnp.float32),
            pltpu.VMEM((_LANES, d), jnp.float32),
            pltpu.VMEM((d, 2, _LANES), jnp.float32),
            pltpu.VMEM((d, 2, _LANES), jnp.float32),
            pltpu.SemaphoreType.DMA,
            pltpu.SemaphoreType.DMA,
            pltpu.SemaphoreType.DMA,
            pltpu.SemaphoreType.DMA,
        ],
        compiler_params=pltpu.CompilerParams(
            use_tc_tiling_on_sc=False,
            needs_layout_passes=False,
        ),
    )
    def gather_kernel(table_hbm, idx_hbm, out_hbm, idx_all,
                      rows_a, rows_b, tr_a, tr_b, sg_a, sg_b, sw_a, sw_b):
        wid = lax.axis_index("s") * 2 + lax.axis_index("c")
        base_u = wid * units_per_w

        pltpu.sync_copy(
            idx_hbm.at[pl.ds(pl.multiple_of(base_u * _LANES, _LANES),
                             units_per_w * _LANES)],
            idx_all)

        def start_gather(i, rows, sem):
            pltpu.async_copy(
                table_hbm.at[idx_all.at[pl.ds(i * _LANES, _LANES)]], rows, sem)

        iota16 = lax.iota(jnp.int32, 16)
        zero16 = jnp.zeros((16,), jnp.int32)
        one16 = jnp.ones((16,), jnp.int32)

        def transpose(rows, tr):
            def tr_body(c0, c2):
                dj = jnp.bitwise_and(c0 + iota16, d - 1)
                for lg in range(_LANES // 16):
                    lv = iota16 + (16 * lg)
                    vec = plsc.load_gather(rows, [lv, dj])
                    plsc.store_scatter(tr, [dj, zero16, lv], -vec)
                    plsc.store_scatter(tr, [dj, one16, lv], vec)
                return c2
            lax.fori_loop(0, d, tr_body, 0, unroll=4)

        def out_window(i):
            u = base_u + i
            s = (u // 64) * 8 + u % 8
            bt = (u // 8) % 8
            return out_hbm.at[s, :, pl.ds(2 * bt, 2), :]

        def start_write(i, tr, sem):
            pltpu.async_copy(tr, out_window(i), sem)

        def wait_gather(rows, sem):
            pltpu.make_async_copy(table_hbm.at[idx_all.at[pl.ds(0, _LANES)]],
                                  rows, sem).wait()

        def wait_write(i, tr, sem):
            pltpu.make_async_copy(tr, out_window(i), sem).wait()

        start_gather(0, rows_a, sg_a)

        def step(k, carry):
            i0 = 2 * k
            i1 = 2 * k + 1
            start_gather(i1, rows_b, sg_b)
            wait_gather(rows_a, sg_a)

            @pl.when(k > 0)
            def _():
                wait_write(i0, tr_a, sw_a)
            transpose(rows_a, tr_a)
            start_write(i0, tr_a, sw_a)

            @pl.when(k < (units_per_w // 2 - 1))
            def _():
                start_gather(i0 + 2, rows_a, sg_a)
            wait_gather(rows_b, sg_b)

            @pl.when(k > 0)
            def _():
                wait_write(i1, tr_b, sw_b)
            transpose(rows_b, tr_b)
            start_write(i1, tr_b, sw_b)
            return carry

        lax.fori_loop(0, units_per_w // 2, step, 0)
        wait_write(0, tr_a, sw_a)
        wait_write(0, tr_b, sw_b)

    return gather_kernel(table, ids_lin)


def _rotate_tc(x_lin, a16, block_s):
    n_s, d, two_bt, lanes = x_lin.shape
    bt = two_bt // 2

    def body(x_ref, a_ref, o_ref):
        v = x_ref[...]
        asgn = a_ref[...]
        t = 2.0 / (1.0 + asgn * asgn)
        c16 = t - 1.0
        ss16 = asgn * t
        i_idx = lax.broadcasted_iota(jnp.int32, v.shape, 2)
        even = (i_idx % 2) == 0
        w = jnp.where(even, jnp.roll(v, -1, axis=2), jnp.roll(v, 1, axis=2))
        o_ref[...] = v * c16 + w * ss16

    return pl.pallas_call(
        body,
        grid=(n_s // block_s,),
        in_specs=[
            pl.BlockSpec((block_s, d, two_bt, lanes), lambda i: (i, 0, 0, 0)),
            pl.BlockSpec((block_s, d, two_bt, lanes), lambda i: (i, 0, 0, 0)),
        ],
        out_specs=pl.BlockSpec((block_s, d, two_bt, lanes),
                               lambda i: (i, 0, 0, 0)),
        out_shape=jax.ShapeDtypeStruct((n_s, d, two_bt, lanes), jnp.float32),
        compiler_params=pltpu.CompilerParams(
            dimension_semantics=("arbitrary",),
        ),
    )(x_lin, a16)


def kernel(x, affix_ids, rotation_params):
    b, s, d, _ = x.shape
    nbt = b // _LANES
    x_lin = (x.transpose(1, 2, 3, 0)
              .reshape(s, d, 2, nbt, _LANES)
              .transpose(0, 1, 3, 2, 4)
              .reshape(s, d, 2 * nbt, _LANES))
    ids_lin = (affix_ids.astype(jnp.int32).T
               .reshape(s // 8, 8, b // _LANES, _LANES)
               .transpose(0, 2, 1, 3)
               .reshape(-1))
    a3 = _gather_transpose_sc(ids_lin, rotation_params)
    out_lin = _rotate_tc(x_lin, a3, block_s=8)
    out = (out_lin.reshape(s, d, nbt, 2, _LANES)
                  .transpose(2, 4, 0, 1, 3)
                  .reshape(b, s, d, 2))
    return out


# speedup vs baseline: 1.2329x; 1.0038x over previous
import functools

import jax
import jax.numpy as jnp
from jax import lax
from jax.experimental import pallas as pl
from jax.experimental.pallas import tpu as pltpu
from jax.experimental.pallas import tpu_sc as plsc

_LANES = 128
_NW = 32
_VOCAB_DIM = 64


def _convert_table_sc(table_t):
    d, v = table_t.shape
    n_tiles = -(-v // _LANES)
    full = v // _LANES
    rem = v - full * _LANES
    pairs = -(-n_tiles // (2 * _NW))
    mesh = plsc.VectorSubcoreMesh(core_axis_name="c", subcore_axis_name="s")

    @functools.partial(
        pl.kernel, mesh=mesh,
        out_type=jax.ShapeDtypeStruct((v * d,), jnp.float32),
        scratch_types=[
            pltpu.VMEM((d, _LANES), jnp.float32),
            pltpu.VMEM((d, _LANES), jnp.float32),
            pltpu.VMEM((_LANES * d,), jnp.float32),
            pltpu.VMEM((_LANES * d,), jnp.float32),
            pltpu.SemaphoreType.DMA,
            pltpu.SemaphoreType.DMA,
            pltpu.SemaphoreType.DMA,
            pltpu.SemaphoreType.DMA,
        ],
        compiler_params=pltpu.CompilerParams(
            use_tc_tiling_on_sc=True, needs_layout_passes=False),
    )
    def conv_kernel(tin, tout, cols_a, cols_b, tr_a, tr_b,
                    si_a, si_b, sw_a, sw_b):
        wid = lax.axis_index("s") * 2 + lax.axis_index("c")
        base_t = wid * 2 * pairs
        iota16 = lax.iota(jnp.int32, 16)

        def start_read(i, cols, sem):
            @pl.when(base_t + i < n_tiles)
            def _():
                pltpu.async_copy(
                    tin.at[:, pl.ds((base_t + i) * _LANES, _LANES)], cols, sem)

        def wait_read(i, cols, sem):
            @pl.when(base_t + i < n_tiles)
            def _():
                pltpu.make_async_copy(
                    tin.at[:, pl.ds(0, _LANES)], cols, sem).wait()

        def transpose(i, cols, tr):
            @pl.when(base_t + i < n_tiles)
            def _():
                def tr_body(c0, c2):
                    dj = jnp.bitwise_and(c0 + iota16, d - 1)
                    for lg in range(_LANES // 16):
                        lv = iota16 + 16 * lg
                        vec = plsc.load_gather(cols, [dj, lv])
                        plsc.store_scatter(tr, [lv * d + dj], vec)
                    return c2
                lax.fori_loop(0, d, tr_body, 0, unroll=4)

        def start_write(i, tr, sem):
            vt = base_t + i

            @pl.when(vt < full)
            def _():
                pltpu.async_copy(
                    tr, tout.at[pl.ds(vt * _LANES * d, _LANES * d)], sem)

            @pl.when(vt == full)
            def _():
                pltpu.sync_copy(tr.at[pl.ds(0, rem * d)],
                                tout.at[pl.ds(full * _LANES * d, rem * d)])

        def wait_write(i, tr, sem):
            @pl.when(base_t + i < full)
            def _():
                pltpu.make_async_copy(
                    tr, tout.at[pl.ds(0, _LANES * d)], sem).wait()

        start_read(0, cols_a, si_a)

        def step(k, carry):
            i0 = 2 * k
            i1 = 2 * k + 1
            start_read(i1, cols_b, si_b)
            wait_read(i0, cols_a, si_a)

            @pl.when(k > 0)
            def _():
                wait_write(i0 - 2, tr_a, sw_a)
            transpose(i0, cols_a, tr_a)
            start_write(i0, tr_a, sw_a)

            @pl.when(k < pairs - 1)
            def _():
                start_read(i0 + 2, cols_a, si_a)
            wait_read(i1, cols_b, si_b)

            @pl.when(k > 0)
            def _():
                wait_write(i1 - 2, tr_b, sw_b)
            transpose(i1, cols_b, tr_b)
            start_write(i1, tr_b, sw_b)
            return carry

        lax.fori_loop(0, pairs, step, 0)
        wait_write(2 * pairs - 2, tr_a, sw_a)
        wait_write(2 * pairs - 1, tr_b, sw_b)

    return conv_kernel(table_t)


def _gather_transpose_sc(ids_lin, table):
    (t_total,) = ids_lin.shape
    _, d = table.shape
    n_units = t_total // _LANES
    units_per_w = n_units // _NW
    mesh = plsc.VectorSubcoreMesh(core_axis_name="c", subcore_axis_name="s")

    @functools.partial(
        pl.kernel,
        mesh=mesh,
        out_type=jax.ShapeDtypeStruct((t_total // (8 * _LANES), d, 16, _LANES),
                                      jnp.float32),
        scratch_types=[
            pltpu.VMEM((units_per_w * _LANES,), jnp.int32),
            pltpu.VMEM((_LANES, d), jnp.float32),
            pltpu.VMEM((_LANES, d), jnp.float32),
            pltpu.VMEM((d, 2, _LANES), jnp.float32),
            pltpu.VMEM((d, 2, _LANES), jnp.float32),
            pltpu.SemaphoreType.DMA,
            pltpu.SemaphoreType.DMA,
            pltpu.SemaphoreType.DMA,
            pltpu.SemaphoreType.DMA,
        ],
        compiler_params=pltpu.CompilerParams(
            use_tc_tiling_on_sc=False,
            needs_layout_passes=False,
        ),
    )
    def gather_kernel(table_hbm, idx_hbm, out_hbm, idx_all,
                      rows_a, rows_b, tr_a, tr_b, sg_a, sg_b, sw_a, sw_b):
        wid = lax.axis_index("s") * 2 + lax.axis_index("c")
        base_u = wid * units_per_w

        pltpu.sync_copy(
            idx_hbm.at[pl.ds(pl.multiple_of(base_u * _LANES, _LANES),
                             units_per_w * _LANES)],
            idx_all)

        def start_gather(i, rows, sem):
            pltpu.async_copy(
                table_hbm.at[idx_all.at[pl.ds(i * _LANES, _LANES)]], rows, sem)

        iota16 = lax.iota(jnp.int32, 16)
        zero16 = jnp.zeros((16,), jnp.int32)
        one16 = jnp.ones((16,), jnp.int32)

        def transpose(rows, tr):
            def tr_body(c0, c2):
                dj = jnp.bitwise_and(c0 + iota16, d - 1)
                for lg in range(_LANES // 16):
                    lv = iota16 + (16 * lg)
                    vec = plsc.load_gather(rows, [lv, dj])
                    plsc.store_scatter(tr, [dj, zero16, lv], -vec)
                    plsc.store_scatter(tr, [dj, one16, lv], vec)
                return c2
            lax.fori_loop(0, d, tr_body, 0, unroll=4)

        def out_window(i):
            u = base_u + i
            s = (u // 64) * 8 + u % 8
            bt = (u // 8) % 8
            return out_hbm.at[s, :, pl.ds(2 * bt, 2), :]

        def start_write(i, tr, sem):
            pltpu.async_copy(tr, out_window(i), sem)

        def wait_gather(rows, sem):
            pltpu.make_async_copy(table_hbm.at[idx_all.at[pl.ds(0, _LANES)]],
                                  rows, sem).wait()

        def wait_write(i, tr, sem):
            pltpu.make_async_copy(tr, out_window(i), sem).wait()

        start_gather(0, rows_a, sg_a)

        def step(k, carry):
            i0 = 2 * k
            i1 = 2 * k + 1
            start_gather(i1, rows_b, sg_b)
            wait_gather(rows_a, sg_a)

            @pl.when(k > 0)
            def _():
                wait_write(i0, tr_a, sw_a)
            transpose(rows_a, tr_a)
            start_write(i0, tr_a, sw_a)

            @pl.when(k < (units_per_w // 2 - 1))
            def _():
                start_gather(i0 + 2, rows_a, sg_a)
            wait_gather(rows_b, sg_b)

            @pl.when(k > 0)
            def _():
                wait_write(i1, tr_b, sw_b)
            transpose(rows_b, tr_b)
            start_write(i1, tr_b, sw_b)
            return carry

        lax.fori_loop(0, units_per_w // 2, step, 0)
        wait_write(0, tr_a, sw_a)
        wait_write(0, tr_b, sw_b)

    return gather_kernel(table, ids_lin)


def _rotate_tc(x_lin, a16, block_s):
    n_s, d, two_bt, lanes = x_lin.shape
    bt = two_bt // 2

    def body(x_ref, a_ref, o_ref):
        v = x_ref[...]
        asgn = a_ref[...]
        t = 2.0 / (1.0 + asgn * asgn)
        c16 = t - 1.0
        ss16 = asgn * t
        i_idx = lax.broadcasted_iota(jnp.int32, v.shape, 2)
        even = (i_idx % 2) == 0
        w = jnp.where(even, jnp.roll(v, -1, axis=2), jnp.roll(v, 1, axis=2))
        o_ref[...] = v * c16 + w * ss16

    return pl.pallas_call(
        body,
        grid=(n_s // block_s,),
        in_specs=[
            pl.BlockSpec((block_s, d, two_bt, lanes), lambda i: (i, 0, 0, 0)),
            pl.BlockSpec((block_s, d, two_bt, lanes), lambda i: (i, 0, 0, 0)),
        ],
        out_specs=pl.BlockSpec((block_s, d, two_bt, lanes),
                               lambda i: (i, 0, 0, 0)),
        out_shape=jax.ShapeDtypeStruct((n_s, d, two_bt, lanes), jnp.float32),
        compiler_params=pltpu.CompilerParams(
            dimension_semantics=("arbitrary",),
        ),
    )(x_lin, a16)


def kernel(x, affix_ids, rotation_params):
    b, s, d, _ = x.shape
    nbt = b // _LANES
    x_lin = (x.transpose(1, 2, 3, 0)
              .reshape(s, d, 2, nbt, _LANES)
              .transpose(0, 1, 3, 2, 4)
              .reshape(s, d, 2 * nbt, _LANES))
    ids_lin = (affix_ids.astype(jnp.int32).T
               .reshape(s // 8, 8, b // _LANES, _LANES)
               .transpose(0, 2, 1, 3)
               .reshape(-1))
    table_lin = _convert_table_sc(rotation_params.T).reshape(
        rotation_params.shape)
    a3 = _gather_transpose_sc(ids_lin, table_lin)
    out_lin = _rotate_tc(x_lin, a3, block_s=8)
    out = (out_lin.reshape(s, d, nbt, 2, _LANES)
                  .transpose(2, 4, 0, 1, 3)
                  .reshape(b, s, d, 2))
    return out
